# baseline (speedup 1.0000x reference)
import functools

import jax
import jax.numpy as jnp
from jax import lax
from jax.experimental import pallas as pl
from jax.experimental.pallas import tpu as pltpu
from jax.experimental.pallas import tpu_sc as plsc

N_EMB = 1000000
D = 64
B = 16384

NC, NS, L = 2, 16, 16
NW = NC * NS
BPW = B // NW
CHUNK = 128
NCHUNK = BPW // CHUNK
GROUPS = BPW // L


def _make_ssq_kernel():
    mesh = plsc.VectorSubcoreMesh(
        core_axis_name="c", subcore_axis_name="s",
        num_cores=NC, num_subcores=NS)

    idx_scratch = [pltpu.VMEM((CHUNK,), jnp.int32) for _ in range(2 * NCHUNK)]

    @functools.partial(
        pl.kernel,
        out_type=jax.ShapeDtypeStruct((B,), jnp.float32),
        mesh=mesh,
        scratch_types=idx_scratch + [
            pltpu.VMEM((BPW, D), jnp.float32),
            pltpu.VMEM((BPW, D), jnp.float32),
            pltpu.VMEM((BPW,), jnp.float32),
            pltpu.SemaphoreType.DMA,
            pltpu.SemaphoreType.DMA,
        ],
        compiler_params=pltpu.CompilerParams(
            needs_layout_passes=False, use_tc_tiling_on_sc=False),
    )
    def ssq_kernel(emb, sid, tid, out, *refs):
        sidx = refs[0:NCHUNK]
        tidx = refs[NCHUNK:2 * NCHUNK]
        srows, trows, ssq, sem_s, sem_t = refs[2 * NCHUNK:]

        wid = lax.axis_index("s") * NC + lax.axis_index("c")
        base = wid * BPW

        copies = []
        for j in range(NCHUNK):
            pltpu.sync_copy(sid.at[pl.ds(base + j * CHUNK, CHUNK)], sidx[j])
            pltpu.sync_copy(tid.at[pl.ds(base + j * CHUNK, CHUNK)], tidx[j])
            copies.append(pltpu.async_copy(
                emb.at[sidx[j]], srows.at[pl.ds(j * CHUNK, CHUNK)], sem_s))
            copies.append(pltpu.async_copy(
                emb.at[tidx[j]], trows.at[pl.ds(j * CHUNK, CHUNK)], sem_t))
        for c in copies:
            c.wait()

        def group_body(g, carry):
            rows = lax.iota(jnp.int32, L) + g * L
            acc = jnp.zeros((L,), jnp.float32)
            for c in range(D):
                cols = jnp.full((L,), c, jnp.int32)
                sv = plsc.load_gather(srows, [rows, cols])
                tv = plsc.load_gather(trows, [rows, cols])
                dv = sv - tv
                acc = acc + dv * dv
            ssq[pl.ds(g * L, L)] = acc
            return carry

        lax.fori_loop(0, GROUPS, group_body, 0)
        pltpu.sync_copy(ssq, out.at[pl.ds(base, BPW)])

    return ssq_kernel


_ssq_call = _make_ssq_kernel()


def _loss_body(ssq_ref, td_ref, cf_ref, out_ref):
    dist = jnp.sqrt(ssq_ref[...]) * 0.125
    err = dist - td_ref[...]
    out_ref[0, 0] = jnp.sum(err * err * cf_ref[...]) * (1.0 / B)


_loss_call = pl.pallas_call(
    _loss_body,
    out_shape=jax.ShapeDtypeStruct((1, 1), jnp.float32),
    out_specs=pl.BlockSpec(memory_space=pltpu.SMEM),
)


def kernel(embeddings, source_id, target_id, target_distance, confidence):
    sid = source_id.astype(jnp.int32)
    tid = target_id.astype(jnp.int32)
    ssq = _ssq_call(embeddings, sid, tid)
    loss = _loss_call(ssq.reshape(128, 128),
                      target_distance.reshape(128, 128),
                      confidence.reshape(128, 128))
    return loss[0, 0]


# speedup vs baseline: 1.5831x; 1.5831x over previous
import functools

import jax
import jax.numpy as jnp
from jax import lax
from jax.experimental import pallas as pl
from jax.experimental.pallas import tpu as pltpu
from jax.experimental.pallas import tpu_sc as plsc

N_EMB = 1000000
D = 64
B = 16384

NC, NS, L = 2, 16, 16
NW = NC * NS
BPW = B // NW
GROUPS = BPW // L


def _make_ssq_kernel():
    mesh = plsc.VectorSubcoreMesh(
        core_axis_name="c", subcore_axis_name="s",
        num_cores=NC, num_subcores=NS)

    @functools.partial(
        pl.kernel,
        out_type=jax.ShapeDtypeStruct((B,), jnp.float32),
        mesh=mesh,
        scratch_types=[
            pltpu.VMEM((BPW,), jnp.int32),
            pltpu.VMEM((BPW,), jnp.int32),
            pltpu.VMEM((L, D), jnp.float32),
            pltpu.VMEM((L, D), jnp.float32),
            pltpu.VMEM((BPW,), jnp.float32),
            pltpu.SemaphoreType.DMA,
            pltpu.SemaphoreType.DMA,
        ],
        compiler_params=pltpu.CompilerParams(
            needs_layout_passes=False, use_tc_tiling_on_sc=True),
    )
    def ssq_kernel(emb, sid, tid, out, sidx_v, tidx_v,
                   sbuf, tbuf, ssq, sem_s, sem_t):
        wid = lax.axis_index("s") * NC + lax.axis_index("c")
        base = wid * BPW

        pltpu.sync_copy(sid.at[pl.ds(base, BPW)], sidx_v)
        pltpu.sync_copy(tid.at[pl.ds(base, BPW)], tidx_v)

        rows16 = lax.iota(jnp.int32, L)

        def group_body(g, carry):
            gbase = g * L
            sids = sidx_v[pl.ds(gbase, L)]
            tids = tidx_v[pl.ds(gbase, L)]
            copies = []
            for j in range(L):
                copies.append(pltpu.async_copy(
                    emb.at[sids[j]], sbuf.at[j], sem_s))
                copies.append(pltpu.async_copy(
                    emb.at[tids[j]], tbuf.at[j], sem_t))
            for c in copies:
                c.wait()
            acc = jnp.zeros((L,), jnp.float32)
            for c in range(D):
                cols = jnp.full((L,), c, jnp.int32)
                sv = plsc.load_gather(sbuf, [rows16, cols])
                tv = plsc.load_gather(tbuf, [rows16, cols])
                dv = sv - tv
                acc = acc + dv * dv
            ssq[pl.ds(gbase, L)] = acc
            return carry

        lax.fori_loop(0, GROUPS, group_body, 0)
        pltpu.sync_copy(ssq, out.at[pl.ds(base, BPW)])

    return ssq_kernel


_ssq_call = _make_ssq_kernel()


def _loss_body(ssq_ref, td_ref, cf_ref, out_ref):
    dist = jnp.sqrt(ssq_ref[...]) * 0.125
    err = dist - td_ref[...]
    out_ref[0, 0] = jnp.sum(err * err * cf_ref[...]) * (1.0 / B)


_loss_call = pl.pallas_call(
    _loss_body,
    out_shape=jax.ShapeDtypeStruct((1, 1), jnp.float32),
    out_specs=pl.BlockSpec(memory_space=pltpu.SMEM),
)


def kernel(embeddings, source_id, target_id, target_distance, confidence):
    sid = source_id.astype(jnp.int32)
    tid = target_id.astype(jnp.int32)
    ssq = _ssq_call(embeddings, sid, tid)
    loss = _loss_call(ssq.reshape(128, 128),
                      target_distance.reshape(128, 128),
                      confidence.reshape(128, 128))
    return loss[0, 0]
